# baseline (speedup 1.0000x reference)
import functools

import jax
import jax.numpy as jnp
from jax import lax
from jax.experimental import pallas as pl
from jax.experimental.pallas import tpu as pltpu
from jax.experimental.pallas import tpu_sc as plsc

N = 10000
D = 128
N_PAD = 10240
NC, NS = 2, 16
NW = NC * NS
C = 128
CA = 56
NB = 6
ROWS_PT = N_PAD // NS
RB = 1000

_mesh = plsc.VectorSubcoreMesh(core_axis_name="c", subcore_axis_name="s")


def _fill(ref, rows, cols, value):
    vec = jnp.full((16,), value, jnp.float32)

    def body(r, _):
        for j in range(cols // 16):
            ref[r, pl.ds(j * 16, 16)] = vec
        return 0

    lax.fori_loop(0, rows, body, 0)


def _zero_spmem_stripe(zbuf, shared, sid, zsem):
    zrows = zbuf.shape[0]
    base = sid * ROWS_PT
    n = ROWS_PT // zrows

    def start(j, _):
        pltpu.async_copy(zbuf, shared.at[pl.ds(base + j * zrows, zrows), :],
                         zsem)
        return 0

    lax.fori_loop(0, n, start, 0)

    def drain(j, _):
        pltpu.make_async_copy(zbuf, shared.at[pl.ds(base, zrows), :],
                              zsem).wait()
        return 0

    lax.fori_loop(0, n, drain, 0)


def _make_sc_degree(G):
    RD = 8
    assert G % RD == 0 and G >= RD

    @functools.partial(
        pl.kernel,
        out_type=jax.ShapeDtypeStruct((NC, N_PAD, D), jnp.float32),
        mesh=_mesh,
        scratch_types=[
            pltpu.VMEM_SHARED((N_PAD, D), jnp.float32),
            pltpu.VMEM((RD, C), jnp.int32),
            pltpu.VMEM((C, D), jnp.float32),
            pltpu.VMEM((8, D), jnp.float32),
            pltpu.SemaphoreType.DMA,
            pltpu.SemaphoreType.DMA((RD,)),
            pltpu.SemaphoreType.DMA((RD,)),
        ],
    )
    def deg_kernel(dst_hbm, out_hbm, deg_sh, dst_b, ones_v, zb,
                   zsem, sdst, ss):
        cid = lax.axis_index("c")
        sid = lax.axis_index("s")
        wid = cid * NS + sid
        _fill(ones_v, C, D, 1.0)
        _fill(zb, 8, D, 0.0)
        _zero_spmem_stripe(zb, deg_sh, sid, zsem)
        plsc.subcore_barrier()

        def load_idx(g, j):
            pltpu.async_copy(dst_hbm.at[wid, g], dst_b.at[j], sdst.at[j])

        for j in range(2):
            load_idx(j, j)

        def body(i, _):
            for j in range(RD):
                g = i * RD + j
                jm6 = (j - 6) % RD

                @pl.when(g >= 6)
                def _wait_scatter():
                    pltpu.make_async_copy(ones_v, deg_sh.at[dst_b.at[jm6]],
                                          ss.at[jm6]).wait()

                @pl.when(g + 2 < G)
                def _load_next():
                    load_idx(g + 2, (j + 2) % RD)

                pltpu.make_async_copy(dst_hbm.at[wid, g], dst_b.at[j],
                                      sdst.at[j]).wait()
                pltpu.async_copy(ones_v, deg_sh.at[dst_b.at[j]],
                                 ss.at[j], add=True)
            return 0

        lax.fori_loop(0, G // RD, body, 0)
        for g in range(G - 6, G):
            j = g % RD
            pltpu.make_async_copy(ones_v, deg_sh.at[dst_b.at[j]],
                                  ss.at[j]).wait()
        plsc.subcore_barrier()
        base = sid * ROWS_PT
        pltpu.sync_copy(deg_sh.at[pl.ds(base, ROWS_PT), :],
                        out_hbm.at[cid, pl.ds(base, ROWS_PT), :])

    return deg_kernel


def _make_sc_edge_agg(G0, G1):
    assert G0 % NB == 0 and G1 % NB == 0 and min(G0, G1) >= NB

    @functools.partial(
        pl.kernel,
        out_type=jax.ShapeDtypeStruct((NC, N_PAD, D), jnp.float32),
        mesh=_mesh,
        scratch_types=[
            pltpu.VMEM_SHARED((N_PAD, D), jnp.float32),
            pltpu.VMEM((NB, CA), jnp.int32),
            pltpu.VMEM((NB, CA), jnp.int32),
            pltpu.VMEM((NB, CA, D), jnp.float32),
            pltpu.VMEM((8, D), jnp.float32),
            pltpu.SemaphoreType.DMA,
            pltpu.SemaphoreType.DMA((NB,)),
            pltpu.SemaphoreType.DMA((NB,)),
            pltpu.SemaphoreType.DMA((NB,)),
            pltpu.SemaphoreType.DMA((NB,)),
        ],
    )
    def agg_kernel(t_hbm, src0_hbm, dst0_hbm, src1_hbm, dst1_hbm, out_hbm,
                   acc_sh, src_b, dst_b, rows, zbuf, zsem, ssrc, sdst, sg, ss):
        cid = lax.axis_index("c")
        sid = lax.axis_index("s")
        _fill(zbuf, 8, D, 0.0)
        _zero_spmem_stripe(zbuf, acc_sh, sid, zsem)
        plsc.subcore_barrier()

        def pipeline(G, src_hbm, dst_hbm):
            def load_idx(g, j):
                pltpu.async_copy(src_hbm.at[sid, g], src_b.at[j], ssrc.at[j])
                pltpu.async_copy(dst_hbm.at[sid, g], dst_b.at[j], sdst.at[j])

            def start_gather(g, j):
                pltpu.make_async_copy(src_hbm.at[sid, g], src_b.at[j],
                                      ssrc.at[j]).wait()
                pltpu.async_copy(t_hbm.at[src_b.at[j]], rows.at[j], sg.at[j])

            for j in range(4):
                load_idx(j, j)
            for j in range(3):
                start_gather(j, j)

            def body(i, _):
                for j in range(NB):
                    g = i * NB + j
                    jm2 = (j - 2) % NB

                    @pl.when(g >= 2)
                    def _wait_scatter():
                        pltpu.make_async_copy(rows.at[jm2],
                                              acc_sh.at[dst_b.at[jm2]],
                                              ss.at[jm2]).wait()

                    @pl.when(g + 4 < G)
                    def _load_next_idx():
                        load_idx(g + 4, (j + 4) % NB)

                    @pl.when(g + 3 < G)
                    def _start_next_gather():
                        start_gather(g + 3, (j + 3) % NB)

                    pltpu.make_async_copy(t_hbm.at[src_b.at[j]], rows.at[j],
                                          sg.at[j]).wait()
                    pltpu.make_async_copy(dst_hbm.at[sid, g], dst_b.at[j],
                                          sdst.at[j]).wait()
                    pltpu.async_copy(rows.at[j], acc_sh.at[dst_b.at[j]],
                                     ss.at[j], add=True)
                return 0

            lax.fori_loop(0, G // NB, body, 0)
            for g in (G - 2, G - 1):
                j = g % NB
                pltpu.make_async_copy(rows.at[j], acc_sh.at[dst_b.at[j]],
                                      ss.at[j]).wait()

        @pl.when(cid == 0)
        def _sc0():
            pipeline(G0, src0_hbm, dst0_hbm)

        @pl.when(cid == 1)
        def _sc1():
            pipeline(G1, src1_hbm, dst1_hbm)

        plsc.subcore_barrier()
        base = sid * ROWS_PT
        pltpu.sync_copy(acc_sh.at[pl.ds(base, ROWS_PT), :],
                        out_hbm.at[cid, pl.ds(base, ROWS_PT), :])

    return agg_kernel


def _dinv_of(deg_ref):
    return lax.rsqrt(deg_ref[0, :, 0] + deg_ref[1, :, 0] + 1.0)


def _tc_mm_body(x_ref, w_ref, o_ref):
    o_ref[...] = jnp.dot(x_ref[...], w_ref[...],
                         preferred_element_type=jnp.float32)


def _tc_scale_body(u_ref, deg_ref, o_ref):
    dinv = _dinv_of(deg_ref)
    o_ref[...] = u_ref[...] * dinv[:, None]


def _tc_mid_body(acc_ref, t_ref, deg_ref, b_ref, w_ref, o_ref):
    dinv = _dinv_of(deg_ref)
    h = (acc_ref[0] + acc_ref[1] + t_ref[...]) * dinv[:, None] + b_ref[...][None, :]
    h = jnp.maximum(h, 0.0)
    o_ref[...] = jnp.dot(h, w_ref[...],
                         preferred_element_type=jnp.float32) * dinv[:, None]


def _tc_last_body(acc_ref, t_ref, deg_ref, b_ref, o_ref):
    dinv = _dinv_of(deg_ref)
    o_ref[...] = ((acc_ref[0] + acc_ref[1] + t_ref[...]) * dinv[:, None]
                  + b_ref[...][None, :])


_rows_spec = pl.BlockSpec((RB, D), lambda i: (i, 0))
_acc_spec = pl.BlockSpec((NC, RB, D), lambda i: (0, i, 0))
_deg_spec = pl.BlockSpec((NC, RB, D), lambda i: (0, i, 0))
_w_spec = pl.BlockSpec((D, D), lambda i: (0, 0))
_b_spec = pl.BlockSpec((D,), lambda i: (0,))
_GRID = (N // RB,)
_OUT = jax.ShapeDtypeStruct((N, D), jnp.float32)

_tc_mm = pl.pallas_call(
    _tc_mm_body, grid=_GRID, out_shape=_OUT,
    in_specs=[_rows_spec, _w_spec], out_specs=_rows_spec)

_tc_scale = pl.pallas_call(
    _tc_scale_body, grid=_GRID, out_shape=_OUT,
    in_specs=[_rows_spec, _deg_spec], out_specs=_rows_spec)

_tc_mid = pl.pallas_call(
    _tc_mid_body, grid=_GRID, out_shape=_OUT,
    in_specs=[_acc_spec, _rows_spec, _deg_spec, _b_spec, _w_spec],
    out_specs=_rows_spec)

_tc_last = pl.pallas_call(
    _tc_last_body, grid=_GRID, out_shape=_OUT,
    in_specs=[_acc_spec, _rows_spec, _deg_spec, _b_spec],
    out_specs=_rows_spec)


def kernel(x, edge_index, W1, b1, W2, b2, W3, b3):
    e = edge_index.shape[1]
    src = edge_index[0].astype(jnp.int32)
    dst = edge_index[1].astype(jnp.int32)
    def _pad_dump(n):
        return N + (jnp.arange(n, dtype=jnp.int32) % (N_PAD - N))

    Gd = -(-e // (NW * C))
    Gd += (-Gd) % 8
    pad_d = NW * Gd * C - e
    dst_d = jnp.concatenate([dst, _pad_dump(pad_d)]).reshape(NW, Gd, C)

    Gtot = -(-e // (NS * CA))
    G0 = (-(-(77 * Gtot) // 100) + NB - 1) // NB * NB
    G1 = max(NB, (Gtot - G0 + NB - 1) // NB * NB)
    n0 = NS * G0 * CA
    n1 = NS * G1 * CA
    pad_a = n0 + n1 - e
    src_p = jnp.concatenate([src, jnp.zeros((pad_a,), jnp.int32)])
    dst_p = jnp.concatenate([dst, _pad_dump(pad_a)])
    src0 = src_p[:n0].reshape(NS, G0, CA)
    dst0 = dst_p[:n0].reshape(NS, G0, CA)
    src1 = src_p[n0:].reshape(NS, G1, CA)
    dst1 = dst_p[n0:].reshape(NS, G1, CA)

    deg2 = _make_sc_degree(Gd)(dst_d)
    agg = _make_sc_edge_agg(G0, G1)

    u1 = _tc_mm(x, W1)
    t1 = _tc_scale(u1, deg2)
    acc1 = agg(t1, src0, dst0, src1, dst1)
    t2 = _tc_mid(acc1, t1, deg2, b1, W2)
    acc2 = agg(t2, src0, dst0, src1, dst1)
    t3 = _tc_mid(acc2, t2, deg2, b2, W3)
    acc3 = agg(t3, src0, dst0, src1, dst1)
    return _tc_last(acc3, t3, deg2, b3)


# speedup vs baseline: 1.0556x; 1.0556x over previous
import functools

import jax
import jax.numpy as jnp
from jax import lax
from jax.experimental import pallas as pl
from jax.experimental.pallas import tpu as pltpu
from jax.experimental.pallas import tpu_sc as plsc

N = 10000
D = 128
N_PAD = 10240
NC, NS = 2, 16
NW = NC * NS
C = 128
CA = 56
NB = 6
ROWS_PT = N_PAD // NS
RB = 1000

_mesh = plsc.VectorSubcoreMesh(core_axis_name="c", subcore_axis_name="s")


def _fill(ref, rows, cols, value):
    vec = jnp.full((16,), value, jnp.float32)

    def body(r, _):
        for j in range(cols // 16):
            ref[r, pl.ds(j * 16, 16)] = vec
        return 0

    lax.fori_loop(0, rows, body, 0)


def _zero_spmem_stripe(zbuf, shared, sid, zsem):
    zrows = zbuf.shape[0]
    base = sid * ROWS_PT
    n = ROWS_PT // zrows

    def start(j, _):
        pltpu.async_copy(zbuf, shared.at[pl.ds(base + j * zrows, zrows), :],
                         zsem)
        return 0

    lax.fori_loop(0, n, start, 0)

    def drain(j, _):
        pltpu.make_async_copy(zbuf, shared.at[pl.ds(base, zrows), :],
                              zsem).wait()
        return 0

    lax.fori_loop(0, n, drain, 0)


def _make_sc_degree(G):
    RD = 8
    assert G % RD == 0 and G >= RD

    @functools.partial(
        pl.kernel,
        out_type=jax.ShapeDtypeStruct((NC, N_PAD, D), jnp.float32),
        mesh=_mesh,
        scratch_types=[
            pltpu.VMEM_SHARED((N_PAD, D), jnp.float32),
            pltpu.VMEM((RD, C), jnp.int32),
            pltpu.VMEM((C, D), jnp.float32),
            pltpu.VMEM((8, D), jnp.float32),
            pltpu.SemaphoreType.DMA,
            pltpu.SemaphoreType.DMA((RD,)),
            pltpu.SemaphoreType.DMA((RD,)),
        ],
    )
    def deg_kernel(dst_hbm, out_hbm, deg_sh, dst_b, ones_v, zb,
                   zsem, sdst, ss):
        cid = lax.axis_index("c")
        sid = lax.axis_index("s")
        wid = cid * NS + sid
        _fill(ones_v, C, D, 1.0)
        _fill(zb, 8, D, 0.0)
        _zero_spmem_stripe(zb, deg_sh, sid, zsem)
        plsc.subcore_barrier()

        def load_idx(g, j):
            pltpu.async_copy(dst_hbm.at[wid, g], dst_b.at[j], sdst.at[j])

        for j in range(2):
            load_idx(j, j)

        def body(i, _):
            for j in range(RD):
                g = i * RD + j
                jm6 = (j - 6) % RD

                @pl.when(g >= 6)
                def _wait_scatter():
                    pltpu.make_async_copy(ones_v, deg_sh.at[dst_b.at[jm6]],
                                          ss.at[jm6]).wait()

                @pl.when(g + 2 < G)
                def _load_next():
                    load_idx(g + 2, (j + 2) % RD)

                pltpu.make_async_copy(dst_hbm.at[wid, g], dst_b.at[j],
                                      sdst.at[j]).wait()
                pltpu.async_copy(ones_v, deg_sh.at[dst_b.at[j]],
                                 ss.at[j], add=True)
            return 0

        lax.fori_loop(0, G // RD, body, 0)
        for g in range(G - 6, G):
            j = g % RD
            pltpu.make_async_copy(ones_v, deg_sh.at[dst_b.at[j]],
                                  ss.at[j]).wait()
        plsc.subcore_barrier()
        base = sid * ROWS_PT
        pltpu.sync_copy(deg_sh.at[pl.ds(base, ROWS_PT), :],
                        out_hbm.at[cid, pl.ds(base, ROWS_PT), :])

    return deg_kernel


def _make_sc_edge_agg(G0, G1):
    assert G0 % NB == 0 and G1 % NB == 0 and min(G0, G1) >= NB

    @functools.partial(
        pl.kernel,
        out_type=jax.ShapeDtypeStruct((NC, N_PAD, D), jnp.float32),
        mesh=_mesh,
        scratch_types=[
            pltpu.VMEM_SHARED((N_PAD, D), jnp.float32),
            pltpu.VMEM((NB, CA), jnp.int32),
            pltpu.VMEM((NB, CA), jnp.int32),
            pltpu.VMEM((NB, CA, D), jnp.float32),
            pltpu.VMEM((8, D), jnp.float32),
            pltpu.SemaphoreType.DMA,
            pltpu.SemaphoreType.DMA((NB,)),
            pltpu.SemaphoreType.DMA((NB,)),
            pltpu.SemaphoreType.DMA((NB,)),
            pltpu.SemaphoreType.DMA((NB,)),
        ],
    )
    def agg_kernel(t_hbm, src0_hbm, dst0_hbm, src1_hbm, dst1_hbm, out_hbm,
                   acc_sh, src_b, dst_b, rows, zbuf, zsem, ssrc, sdst, sg, ss):
        cid = lax.axis_index("c")
        sid = lax.axis_index("s")
        _fill(zbuf, 8, D, 0.0)
        _zero_spmem_stripe(zbuf, acc_sh, sid, zsem)
        plsc.subcore_barrier()

        def pipeline(G, src_hbm, dst_hbm):
            def load_idx(g, j):
                pltpu.async_copy(src_hbm.at[sid, g], src_b.at[j], ssrc.at[j])
                pltpu.async_copy(dst_hbm.at[sid, g], dst_b.at[j], sdst.at[j])

            def start_gather(g, j):
                pltpu.make_async_copy(src_hbm.at[sid, g], src_b.at[j],
                                      ssrc.at[j]).wait()
                pltpu.async_copy(t_hbm.at[src_b.at[j]], rows.at[j], sg.at[j])

            for j in range(4):
                load_idx(j, j)
            for j in range(3):
                start_gather(j, j)

            def body(i, _):
                for j in range(NB):
                    g = i * NB + j
                    jm2 = (j - 2) % NB

                    @pl.when(g >= 2)
                    def _wait_scatter():
                        pltpu.make_async_copy(rows.at[jm2],
                                              acc_sh.at[dst_b.at[jm2]],
                                              ss.at[jm2]).wait()

                    @pl.when(g + 4 < G)
                    def _load_next_idx():
                        load_idx(g + 4, (j + 4) % NB)

                    @pl.when(g + 3 < G)
                    def _start_next_gather():
                        start_gather(g + 3, (j + 3) % NB)

                    pltpu.make_async_copy(t_hbm.at[src_b.at[j]], rows.at[j],
                                          sg.at[j]).wait()
                    pltpu.make_async_copy(dst_hbm.at[sid, g], dst_b.at[j],
                                          sdst.at[j]).wait()
                    pltpu.async_copy(rows.at[j], acc_sh.at[dst_b.at[j]],
                                     ss.at[j], add=True)
                return 0

            lax.fori_loop(0, G // NB, body, 0)
            for g in (G - 2, G - 1):
                j = g % NB
                pltpu.make_async_copy(rows.at[j], acc_sh.at[dst_b.at[j]],
                                      ss.at[j]).wait()

        @pl.when(cid == 0)
        def _sc0():
            pipeline(G0, src0_hbm, dst0_hbm)

        @pl.when(cid == 1)
        def _sc1():
            pipeline(G1, src1_hbm, dst1_hbm)

        plsc.subcore_barrier()
        base = sid * ROWS_PT
        pltpu.sync_copy(acc_sh.at[pl.ds(base, ROWS_PT), :],
                        out_hbm.at[cid, pl.ds(base, ROWS_PT), :])

    return agg_kernel


def _dinv_of(deg_ref):
    return lax.rsqrt(deg_ref[0, :, 0] + deg_ref[1, :, 0] + 1.0)


def _tc_mm_body(x_ref, w_ref, o_ref):
    o_ref[...] = jnp.dot(x_ref[...], w_ref[...],
                         preferred_element_type=jnp.float32)


def _tc_scale_body(u_ref, deg_ref, o_ref):
    dinv = _dinv_of(deg_ref)
    o_ref[...] = u_ref[...] * dinv[:, None]


def _tc_mid_body(acc_ref, t_ref, deg_ref, b_ref, w_ref, o_ref):
    dinv = _dinv_of(deg_ref)
    h = (acc_ref[0] + acc_ref[1] + t_ref[...]) * dinv[:, None] + b_ref[...][None, :]
    h = jnp.maximum(h, 0.0)
    o_ref[...] = jnp.dot(h, w_ref[...],
                         preferred_element_type=jnp.float32) * dinv[:, None]


def _tc_last_body(acc_ref, t_ref, deg_ref, b_ref, o_ref):
    dinv = _dinv_of(deg_ref)
    o_ref[...] = ((acc_ref[0] + acc_ref[1] + t_ref[...]) * dinv[:, None]
                  + b_ref[...][None, :])


_rows_spec = pl.BlockSpec((RB, D), lambda i: (i, 0))
_acc_spec = pl.BlockSpec((NC, RB, D), lambda i: (0, i, 0))
_deg_spec = pl.BlockSpec((NC, RB, D), lambda i: (0, i, 0))
_w_spec = pl.BlockSpec((D, D), lambda i: (0, 0))
_b_spec = pl.BlockSpec((D,), lambda i: (0,))
_GRID = (N // RB,)
_OUT = jax.ShapeDtypeStruct((N, D), jnp.float32)

_tc_mm = pl.pallas_call(
    _tc_mm_body, grid=_GRID, out_shape=_OUT,
    in_specs=[_rows_spec, _w_spec], out_specs=_rows_spec)

_tc_scale = pl.pallas_call(
    _tc_scale_body, grid=_GRID, out_shape=_OUT,
    in_specs=[_rows_spec, _deg_spec], out_specs=_rows_spec)

_tc_mid = pl.pallas_call(
    _tc_mid_body, grid=_GRID, out_shape=_OUT,
    in_specs=[_acc_spec, _rows_spec, _deg_spec, _b_spec, _w_spec],
    out_specs=_rows_spec)

_tc_last = pl.pallas_call(
    _tc_last_body, grid=_GRID, out_shape=_OUT,
    in_specs=[_acc_spec, _rows_spec, _deg_spec, _b_spec],
    out_specs=_rows_spec)


def kernel(x, edge_index, W1, b1, W2, b2, W3, b3):
    e = edge_index.shape[1]
    src = edge_index[0].astype(jnp.int32)
    dst = edge_index[1].astype(jnp.int32)
    def _pad_dump(n):
        return N + (jnp.arange(n, dtype=jnp.int32) % (N_PAD - N))

    Gd = -(-e // (NW * C))
    Gd += (-Gd) % 8
    pad_d = NW * Gd * C - e
    dst_d = jnp.concatenate([dst, _pad_dump(pad_d)]).reshape(NW, Gd, C)

    Gtot = -(-e // (NS * CA))
    G0 = (-(-(88 * Gtot) // 100) + NB - 1) // NB * NB
    G1 = max(NB, (Gtot - G0 + NB - 1) // NB * NB)
    n0 = NS * G0 * CA
    n1 = NS * G1 * CA
    pad_a = n0 + n1 - e
    src_p = jnp.concatenate([src, jnp.zeros((pad_a,), jnp.int32)])
    dst_p = jnp.concatenate([dst, _pad_dump(pad_a)])
    src0 = src_p[:n0].reshape(NS, G0, CA)
    dst0 = dst_p[:n0].reshape(NS, G0, CA)
    src1 = src_p[n0:].reshape(NS, G1, CA)
    dst1 = dst_p[n0:].reshape(NS, G1, CA)

    deg2 = _make_sc_degree(Gd)(dst_d)
    agg = _make_sc_edge_agg(G0, G1)

    u1 = _tc_mm(x, W1)
    t1 = _tc_scale(u1, deg2)
    acc1 = agg(t1, src0, dst0, src1, dst1)
    t2 = _tc_mid(acc1, t1, deg2, b1, W2)
    acc2 = agg(t2, src0, dst0, src1, dst1)
    t3 = _tc_mid(acc2, t2, deg2, b2, W3)
    acc3 = agg(t3, src0, dst0, src1, dst1)
    return _tc_last(acc3, t3, deg2, b3)
